# baseline (speedup 1.0000x reference)
import functools

import jax
import jax.numpy as jnp
from jax import lax
from jax.experimental import pallas as pl
from jax.experimental.pallas import tpu as pltpu
from jax.experimental.pallas import tpu_sc as plsc

N = 10000
NP = 10240
E = 320000
D_IN = 128
D_HID = 256
D_OUT = 128
N_EXP = 4

NTILES = 32
NB = 80
ET = NB * 128
EP = NTILES * ET
DEG_W = 128
RPT = NP // 16

BLK = 256

_MESH = plsc.VectorSubcoreMesh(core_axis_name="c", subcore_axis_name="s")


@functools.partial(
    pl.kernel,
    mesh=_MESH,
    out_type=jax.ShapeDtypeStruct((2 * NP, DEG_W), jnp.float32),
    scratch_types=[
        pltpu.VMEM((NB, 128), jnp.int32),
        pltpu.VMEM((128, DEG_W), jnp.float32),
        pltpu.VMEM_SHARED((NP, DEG_W), jnp.float32),
    ],
)
def _deg_kernel(dst_hbm, ones_hbm, zeros_hbm, out_hbm, dst_v, ones_v, acc):
    c = lax.axis_index("c")
    s = lax.axis_index("s")
    wid = s * 2 + c
    pltpu.sync_copy(zeros_hbm.at[pl.ds(s * RPT, RPT)],
                    acc.at[pl.ds(s * RPT, RPT)])
    pltpu.sync_copy(ones_hbm, ones_v)
    pltpu.sync_copy(dst_hbm.at[wid], dst_v)
    plsc.subcore_barrier()

    def body(j, carry):
        pltpu.sync_copy(ones_v, acc.at[dst_v.at[j]], add=True)
        return carry

    lax.fori_loop(0, NB, body, 0)
    plsc.subcore_barrier()
    pltpu.sync_copy(acc.at[pl.ds(s * RPT, RPT)],
                    out_hbm.at[pl.ds(c * NP + s * RPT, RPT)])


@functools.partial(
    pl.kernel,
    mesh=_MESH,
    out_type=jax.ShapeDtypeStruct((2 * NP, 128), jnp.float32),
    scratch_types=[
        pltpu.VMEM((NB, 128), jnp.int32),
        pltpu.VMEM((NB, 128), jnp.int32),
        pltpu.VMEM((128, 128), jnp.float32),
        pltpu.VMEM_SHARED((NP, 128), jnp.float32),
        pltpu.SemaphoreType.DMA,
    ],
)
def _prop_kernel(table_hbm, src_hbm, dst_hbm, zeros_hbm, out_hbm,
                 src_v, dst_v, rows_v, acc, sem):
    c = lax.axis_index("c")
    s = lax.axis_index("s")
    wid = s * 2 + c
    pltpu.sync_copy(zeros_hbm.at[pl.ds(s * RPT, RPT)],
                    acc.at[pl.ds(s * RPT, RPT)])
    pltpu.sync_copy(src_hbm.at[wid], src_v)
    pltpu.sync_copy(dst_hbm.at[wid], dst_v)
    plsc.subcore_barrier()

    def body(j, carry):
        pltpu.async_copy(table_hbm.at[src_v.at[j]], rows_v, sem).wait()
        pltpu.sync_copy(rows_v, acc.at[dst_v.at[j]], add=True)
        return carry

    lax.fori_loop(0, NB, body, 0)
    plsc.subcore_barrier()
    pltpu.sync_copy(acc.at[pl.ds(s * RPT, RPT)],
                    out_hbm.at[pl.ds(c * NP + s * RPT, RPT)])


def _prep_body(x_ref, d0_ref, d1_ref, xs_ref, inv_ref):
    d = d0_ref[:, 0:1] + d1_ref[:, 0:1]
    inv = lax.rsqrt(jnp.maximum(d, 1.0))
    inv_b = jnp.broadcast_to(inv, (BLK, 128))
    inv_ref[...] = inv_b
    xs_ref[...] = x_ref[...] * inv_b


def _prep_call(x_pad, deg0, deg1):
    return pl.pallas_call(
        _prep_body,
        grid=(NP // BLK,),
        in_specs=[
            pl.BlockSpec((BLK, 128), lambda i: (i, 0)),
            pl.BlockSpec((BLK, DEG_W), lambda i: (i, 0)),
            pl.BlockSpec((BLK, DEG_W), lambda i: (i, 0)),
        ],
        out_specs=[
            pl.BlockSpec((BLK, 128), lambda i: (i, 0)),
            pl.BlockSpec((BLK, 128), lambda i: (i, 0)),
        ],
        out_shape=[
            jax.ShapeDtypeStruct((NP, 128), jnp.float32),
            jax.ShapeDtypeStruct((NP, 128), jnp.float32),
        ],
    )(x_pad, deg0, deg1)


def _experts_body(s0_ref, s1_ref, inv_ref, w1_ref, b1_ref, w2_ref, y4_ref):
    inv = inv_ref[...]
    px = (s0_ref[...] + s1_ref[...]) * inv
    for e in range(N_EXP):
        h = jnp.dot(px, w1_ref[e], preferred_element_type=jnp.float32)
        h = jnp.maximum(h + b1_ref[e:e + 1, :], 0.0)
        y = jnp.dot(h, w2_ref[e], preferred_element_type=jnp.float32)
        y4_ref[e] = y * inv


def _experts_call(s0, s1, inv_bc, W1, b1, W2):
    return pl.pallas_call(
        _experts_body,
        grid=(NP // BLK,),
        in_specs=[
            pl.BlockSpec((BLK, 128), lambda i: (i, 0)),
            pl.BlockSpec((BLK, 128), lambda i: (i, 0)),
            pl.BlockSpec((BLK, 128), lambda i: (i, 0)),
            pl.BlockSpec((N_EXP, D_IN, D_HID), lambda i: (0, 0, 0)),
            pl.BlockSpec((N_EXP, D_HID), lambda i: (0, 0)),
            pl.BlockSpec((N_EXP, D_HID, D_OUT), lambda i: (0, 0, 0)),
        ],
        out_specs=pl.BlockSpec((N_EXP, BLK, 128), lambda i: (0, i, 0)),
        out_shape=jax.ShapeDtypeStruct((N_EXP, NP, 128), jnp.float32),
    )(s0, s1, inv_bc, W1, b1, W2)


def _combine_body(x_ref, wg_ref, thr_ref, msk_ref, b2_ref, inv_ref, t_ref,
                  out_ref):
    logits = jnp.dot(x_ref[...], wg_ref[...],
                     preferred_element_type=jnp.float32)
    col = lax.broadcasted_iota(jnp.int32, (BLK, 128), 1)
    valid = col < N_EXP
    lm = jnp.where(valid, logits, -1e30)
    m = jnp.max(lm, axis=1, keepdims=True)
    ex = jnp.where(valid, jnp.exp(logits - m), 0.0)
    soft = ex / jnp.sum(ex, axis=1, keepdims=True)
    hard = 0.5 * (jnp.sign(logits - thr_ref[0:1, :]) + 1.0)
    g = soft * hard * msk_ref[0:1, :]
    g = g / (jnp.sum(g, axis=1, keepdims=True) + 1e-10)

    out = jnp.dot(g, b2_ref[...], preferred_element_type=jnp.float32)
    inv = inv_ref[...]
    for e in range(N_EXP):
        pe = (t_ref[2 * e] + t_ref[2 * e + 1]) * inv
        out = out + g[:, e:e + 1] * pe
    out_ref[...] = out


def _combine_call(x_pad, wg_pad, thr_bc, msk_bc, b2_pad, inv_bc, tstack):
    return pl.pallas_call(
        _combine_body,
        grid=(NP // BLK,),
        in_specs=[
            pl.BlockSpec((BLK, 128), lambda i: (i, 0)),
            pl.BlockSpec((128, 128), lambda i: (0, 0)),
            pl.BlockSpec((8, 128), lambda i: (0, 0)),
            pl.BlockSpec((8, 128), lambda i: (0, 0)),
            pl.BlockSpec((128, 128), lambda i: (0, 0)),
            pl.BlockSpec((BLK, 128), lambda i: (i, 0)),
            pl.BlockSpec((2 * N_EXP, BLK, 128), lambda i: (0, i, 0)),
        ],
        out_specs=pl.BlockSpec((BLK, 128), lambda i: (i, 0)),
        out_shape=jax.ShapeDtypeStruct((NP, 128), jnp.float32),
    )(x_pad, wg_pad, thr_bc, msk_bc, b2_pad, inv_bc, tstack)


def kernel(x, edge_index, w_gate, gate_threshold, W1, b1, W2, b2,
           experts_mask):
    f32 = jnp.float32
    x = x.astype(f32)
    src = edge_index[0].astype(jnp.int32)
    dst = edge_index[1].astype(jnp.int32)
    padn = EP - E
    fill = jnp.full((padn,), N, jnp.int32)
    src_p = jnp.concatenate([src, fill]).reshape(NTILES, NB, 128)
    dst_p = jnp.concatenate([dst, fill]).reshape(NTILES, NB, 128)
    x_pad = jnp.pad(x, ((0, NP - N), (0, 0)))
    zeros128 = jnp.zeros((NP, 128), f32)
    zerosW = jnp.zeros((NP, DEG_W), f32)
    onesW = jnp.ones((128, DEG_W), f32)

    degp = _deg_kernel(dst_p, onesW, zerosW)
    xs, inv_bc = _prep_call(x_pad, degp[:NP], degp[NP:])
    sp = _prop_kernel(xs, src_p, dst_p, zeros128)
    y4 = _experts_call(sp[:NP], sp[NP:], inv_bc,
                       W1.astype(f32), b1.astype(f32), W2.astype(f32))
    t = [_prop_kernel(y4[e], src_p, dst_p, zeros128) for e in range(N_EXP)]
    tstack = jnp.stack([t[e][h * NP:(h + 1) * NP]
                        for e in range(N_EXP) for h in (0, 1)])

    wg_pad = jnp.zeros((128, 128), f32).at[:, :N_EXP].set(w_gate.astype(f32))
    thr_bc = jnp.zeros((8, 128), f32).at[:, :N_EXP].set(
        jnp.broadcast_to(gate_threshold.astype(f32), (8, N_EXP)))
    msk_bc = jnp.zeros((8, 128), f32).at[:, :N_EXP].set(
        jnp.broadcast_to(experts_mask.astype(f32), (8, N_EXP)))
    b2_pad = jnp.zeros((128, 128), f32).at[:N_EXP, :].set(b2.astype(f32))

    out = _combine_call(x_pad, wg_pad, thr_bc, msk_bc, b2_pad, inv_bc, tstack)
    return out[:N]


# speedup vs baseline: 2.4464x; 2.4464x over previous
import functools

import jax
import jax.numpy as jnp
from jax import lax
from jax.experimental import pallas as pl
from jax.experimental.pallas import tpu as pltpu
from jax.experimental.pallas import tpu_sc as plsc

N = 10000
NP = 10240
E = 320000
D_IN = 128
D_HID = 256
D_OUT = 128
N_EXP = 4

NTILES = 32
NB = 80
ET = NB * 128
SW = 2
SB = NB // SW
EP = NTILES * ET
DEG_W = 128
RPT = NP // 16

BLK = 256

_MESH = plsc.VectorSubcoreMesh(core_axis_name="c", subcore_axis_name="s")


@functools.partial(
    pl.kernel,
    mesh=_MESH,
    out_type=jax.ShapeDtypeStruct((2 * NP, DEG_W), jnp.float32),
    scratch_types=[
        pltpu.VMEM((NB, 128), jnp.int32),
        pltpu.VMEM((128, DEG_W), jnp.float32),
        pltpu.VMEM_SHARED((NP, DEG_W), jnp.float32),
        pltpu.SemaphoreType.DMA,
    ],
)
def _deg_kernel(dst_hbm, ones_hbm, zeros_hbm, out_hbm, dst_v, ones_v, acc,
                sem):
    c = lax.axis_index("c")
    s = lax.axis_index("s")
    wid = s * 2 + c
    pltpu.sync_copy(zeros_hbm.at[pl.ds(s * RPT, RPT)],
                    acc.at[pl.ds(s * RPT, RPT)])
    pltpu.sync_copy(ones_hbm, ones_v)
    pltpu.sync_copy(dst_hbm.at[wid], dst_v)
    plsc.subcore_barrier()

    def body(i, carry):
        for b in range(2):
            pltpu.async_copy(ones_v, acc.at[dst_v.at[2 * i + b]], sem,
                             add=True)
        for b in range(2):
            pltpu.make_async_copy(ones_v, acc.at[dst_v.at[2 * i + b]],
                                  sem).wait()
        return carry

    lax.fori_loop(0, NB // 2, body, 0)
    plsc.subcore_barrier()
    pltpu.sync_copy(acc.at[pl.ds(s * RPT, RPT)],
                    out_hbm.at[pl.ds(c * NP + s * RPT, RPT)])


def _make_prop(nchunks):
    @functools.partial(
        pl.kernel,
        mesh=_MESH,
        out_type=jax.ShapeDtypeStruct((nchunks * 2 * NP, 128), jnp.float32),
        scratch_types=[
            pltpu.VMEM((NB, 128), jnp.int32),
            pltpu.VMEM((NB, 128), jnp.int32),
            pltpu.VMEM((128, 128), jnp.float32),
            pltpu.VMEM_SHARED((NP, 128), jnp.float32),
            pltpu.SemaphoreType.DMA,
        ],
    )
    def prop(table_hbm, src_hbm, dst_hbm, zeros_hbm, out_hbm,
             src_v, dst_v, rows_v, acc, sem):
        c = lax.axis_index("c")
        s = lax.axis_index("s")
        wid = s * 2 + c
        pltpu.sync_copy(dst_hbm.at[wid], dst_v)
        for ci in range(nchunks):
            pltpu.sync_copy(zeros_hbm.at[pl.ds(s * RPT, RPT)],
                            acc.at[pl.ds(s * RPT, RPT)])
            pltpu.sync_copy(src_hbm.at[ci * NTILES + wid], src_v)
            plsc.subcore_barrier()

            def body(j, carry):
                pltpu.async_copy(table_hbm.at[src_v.at[j]], rows_v,
                                 sem).wait()
                pltpu.sync_copy(rows_v, acc.at[dst_v.at[j]], add=True)
                return carry

            lax.fori_loop(0, NB, body, 0)
            plsc.subcore_barrier()
            pltpu.sync_copy(
                acc.at[pl.ds(s * RPT, RPT)],
                out_hbm.at[pl.ds((2 * ci + c) * NP + s * RPT, RPT)])

    return prop


_prop1 = _make_prop(1)
_prop4 = _make_prop(N_EXP)


def _prep_body(x_ref, d0_ref, d1_ref, xs_ref, inv_ref):
    d = d0_ref[:, 0:1] + d1_ref[:, 0:1]
    inv = lax.rsqrt(jnp.maximum(d, 1.0))
    inv_b = jnp.broadcast_to(inv, (BLK, 128))
    inv_ref[...] = inv_b
    xs_ref[...] = x_ref[...] * inv_b


def _prep_call(x_pad, deg0, deg1):
    return pl.pallas_call(
        _prep_body,
        grid=(NP // BLK,),
        in_specs=[
            pl.BlockSpec((BLK, 128), lambda i: (i, 0)),
            pl.BlockSpec((BLK, DEG_W), lambda i: (i, 0)),
            pl.BlockSpec((BLK, DEG_W), lambda i: (i, 0)),
        ],
        out_specs=[
            pl.BlockSpec((BLK, 128), lambda i: (i, 0)),
            pl.BlockSpec((BLK, 128), lambda i: (i, 0)),
        ],
        out_shape=[
            jax.ShapeDtypeStruct((NP, 128), jnp.float32),
            jax.ShapeDtypeStruct((NP, 128), jnp.float32),
        ],
    )(x_pad, deg0, deg1)


def _experts_body(s0_ref, s1_ref, inv_ref, w1_ref, b1_ref, w2_ref, y4_ref):
    inv = inv_ref[...]
    px = (s0_ref[...] + s1_ref[...]) * inv
    for e in range(N_EXP):
        h = jnp.dot(px, w1_ref[e], preferred_element_type=jnp.float32)
        h = jnp.maximum(h + b1_ref[e:e + 1, :], 0.0)
        y = jnp.dot(h, w2_ref[e], preferred_element_type=jnp.float32)
        y4_ref[e] = y * inv


def _experts_call(s0, s1, inv_bc, W1, b1, W2):
    return pl.pallas_call(
        _experts_body,
        grid=(NP // BLK,),
        in_specs=[
            pl.BlockSpec((BLK, 128), lambda i: (i, 0)),
            pl.BlockSpec((BLK, 128), lambda i: (i, 0)),
            pl.BlockSpec((BLK, 128), lambda i: (i, 0)),
            pl.BlockSpec((N_EXP, D_IN, D_HID), lambda i: (0, 0, 0)),
            pl.BlockSpec((N_EXP, D_HID), lambda i: (0, 0)),
            pl.BlockSpec((N_EXP, D_HID, D_OUT), lambda i: (0, 0, 0)),
        ],
        out_specs=pl.BlockSpec((N_EXP, BLK, 128), lambda i: (0, i, 0)),
        out_shape=jax.ShapeDtypeStruct((N_EXP, NP, 128), jnp.float32),
    )(s0, s1, inv_bc, W1, b1, W2)


def _combine_body(x_ref, wg_ref, thr_ref, msk_ref, b2_ref, inv_ref, t_ref,
                  out_ref):
    logits = jnp.dot(x_ref[...], wg_ref[...],
                     preferred_element_type=jnp.float32)
    col = lax.broadcasted_iota(jnp.int32, (BLK, 128), 1)
    valid = col < N_EXP
    lm = jnp.where(valid, logits, -1e30)
    m = jnp.max(lm, axis=1, keepdims=True)
    ex = jnp.where(valid, jnp.exp(logits - m), 0.0)
    soft = ex / jnp.sum(ex, axis=1, keepdims=True)
    hard = 0.5 * (jnp.sign(logits - thr_ref[0:1, :]) + 1.0)
    g = soft * hard * msk_ref[0:1, :]
    g = g / (jnp.sum(g, axis=1, keepdims=True) + 1e-10)

    out = jnp.dot(g, b2_ref[...], preferred_element_type=jnp.float32)
    inv = inv_ref[...]
    for e in range(N_EXP):
        pe = (t_ref[2 * e] + t_ref[2 * e + 1]) * inv
        out = out + g[:, e:e + 1] * pe
    out_ref[...] = out


def _combine_call(x_pad, wg_pad, thr_bc, msk_bc, b2_pad, inv_bc, tstack):
    return pl.pallas_call(
        _combine_body,
        grid=(NP // BLK,),
        in_specs=[
            pl.BlockSpec((BLK, 128), lambda i: (i, 0)),
            pl.BlockSpec((128, 128), lambda i: (0, 0)),
            pl.BlockSpec((8, 128), lambda i: (0, 0)),
            pl.BlockSpec((8, 128), lambda i: (0, 0)),
            pl.BlockSpec((128, 128), lambda i: (0, 0)),
            pl.BlockSpec((BLK, 128), lambda i: (i, 0)),
            pl.BlockSpec((2 * N_EXP, BLK, 128), lambda i: (0, i, 0)),
        ],
        out_specs=pl.BlockSpec((BLK, 128), lambda i: (i, 0)),
        out_shape=jax.ShapeDtypeStruct((NP, 128), jnp.float32),
    )(x_pad, wg_pad, thr_bc, msk_bc, b2_pad, inv_bc, tstack)


def kernel(x, edge_index, w_gate, gate_threshold, W1, b1, W2, b2,
           experts_mask):
    f32 = jnp.float32
    x = x.astype(f32)
    src = edge_index[0].astype(jnp.int32)
    dst = edge_index[1].astype(jnp.int32)
    padn = EP - E
    fill = N + (jnp.arange(padn, dtype=jnp.int32) % (NP - N))
    src_flat = jnp.concatenate([src, fill])
    dst_flat = jnp.concatenate([dst, fill])
    dst_p = dst_flat.reshape(NTILES, NB, 128)
    src_p = src_flat.reshape(NTILES, NB, 128)
    shift = (jnp.arange(N_EXP, dtype=jnp.int32) * NP)[:, None]
    src4 = (src_flat[None, :] + shift).reshape(N_EXP * NTILES, NB, 128)
    x_pad = jnp.pad(x, ((0, NP - N), (0, 0)))
    zeros128 = jnp.zeros((NP, 128), f32)
    zerosW = jnp.zeros((NP, DEG_W), f32)
    onesW = jnp.ones((128, DEG_W), f32)

    degp = _deg_kernel(dst_p, onesW, zerosW)
    xs, inv_bc = _prep_call(x_pad, degp[:NP], degp[NP:])
    sp = _prop1(xs, src_p, dst_p, zeros128)
    y4 = _experts_call(sp[:NP], sp[NP:], inv_bc,
                       W1.astype(f32), b1.astype(f32), W2.astype(f32))
    t4 = _prop4(y4.reshape(N_EXP * NP, 128), src4, dst_p, zeros128)
    tstack = t4.reshape(2 * N_EXP, NP, 128)

    wg_pad = jnp.zeros((128, 128), f32).at[:, :N_EXP].set(w_gate.astype(f32))
    thr_bc = jnp.zeros((8, 128), f32).at[:, :N_EXP].set(
        jnp.broadcast_to(gate_threshold.astype(f32), (8, N_EXP)))
    msk_bc = jnp.zeros((8, 128), f32).at[:, :N_EXP].set(
        jnp.broadcast_to(experts_mask.astype(f32), (8, N_EXP)))
    b2_pad = jnp.zeros((128, 128), f32).at[:N_EXP, :].set(b2.astype(f32))

    out = _combine_call(x_pad, wg_pad, thr_bc, msk_bc, b2_pad, inv_bc, tstack)
    return out[:N]


# speedup vs baseline: 3.0561x; 1.2492x over previous
import functools

import jax
import jax.numpy as jnp
from jax import lax
from jax.experimental import pallas as pl
from jax.experimental.pallas import tpu as pltpu
from jax.experimental.pallas import tpu_sc as plsc

N = 10000
NP = 10240
E = 320000
D_IN = 128
D_HID = 256
D_OUT = 128
N_EXP = 4

NTILES = 32
NB = 80
ET = NB * 128
SW = 2
SB = NB // SW
EP = NTILES * ET
DEG_W = 128
RPT = NP // 16

BLK = 256

_MESH = plsc.VectorSubcoreMesh(core_axis_name="c", subcore_axis_name="s")


@functools.partial(
    pl.kernel,
    mesh=_MESH,
    out_type=jax.ShapeDtypeStruct((2 * NP, DEG_W), jnp.float32),
    scratch_types=[
        pltpu.VMEM((NB, 128), jnp.int32),
        pltpu.VMEM((128, DEG_W), jnp.float32),
        pltpu.VMEM_SHARED((NP, DEG_W), jnp.float32),
        pltpu.SemaphoreType.DMA,
    ],
)
def _deg_kernel(dst_hbm, ones_hbm, zeros_hbm, out_hbm, dst_v, ones_v, acc,
                sem):
    c = lax.axis_index("c")
    s = lax.axis_index("s")
    wid = s * 2 + c
    pltpu.sync_copy(zeros_hbm.at[pl.ds(s * RPT, RPT)],
                    acc.at[pl.ds(s * RPT, RPT)])
    pltpu.sync_copy(ones_hbm, ones_v)
    pltpu.sync_copy(dst_hbm.at[wid], dst_v)
    plsc.subcore_barrier()

    def body(i, carry):
        for b in range(2):
            pltpu.async_copy(ones_v, acc.at[dst_v.at[2 * i + b]], sem,
                             add=True)
        for b in range(2):
            pltpu.make_async_copy(ones_v, acc.at[dst_v.at[2 * i + b]],
                                  sem).wait()
        return carry

    lax.fori_loop(0, NB // 2, body, 0)
    plsc.subcore_barrier()
    pltpu.sync_copy(acc.at[pl.ds(s * RPT, RPT)],
                    out_hbm.at[pl.ds(c * NP + s * RPT, RPT)])


def _make_prop(nchunks):
    @functools.partial(
        pl.kernel,
        mesh=_MESH,
        out_type=jax.ShapeDtypeStruct((nchunks * 2 * NP, 128), jnp.float32),
        scratch_types=[
            pltpu.VMEM((SB, 128), jnp.int32),
            pltpu.VMEM((SB, 128), jnp.int32),
            pltpu.VMEM((128, 128), jnp.float32),
            pltpu.VMEM((128, 128), jnp.float32),
            pltpu.VMEM_SHARED((NP, 128), jnp.float32),
            pltpu.SemaphoreType.DMA,
            pltpu.SemaphoreType.DMA,
            pltpu.SemaphoreType.DMA,
        ],
    )
    def prop(table_hbm, src_hbm, dst_hbm, zeros_hbm, out_hbm,
             src_v, dst_v, r0, r1, acc, sg, ss0, ss1):
        c = lax.axis_index("c")
        s = lax.axis_index("s")
        wid = s * 2 + c
        rows = [r0, r1]
        ss = [ss0, ss1]

        def gather_sync(j, b):
            pltpu.async_copy(table_hbm.at[src_v.at[j]], rows[b], sg).wait()

        def scatter(j, b):
            pltpu.async_copy(rows[b], acc.at[dst_v.at[j]], ss[b], add=True)

        def wait_scatter(j, b):
            pltpu.make_async_copy(rows[b], acc.at[dst_v.at[j]], ss[b]).wait()

        for ci in range(nchunks):
            pltpu.sync_copy(zeros_hbm.at[pl.ds(s * RPT, RPT)],
                            acc.at[pl.ds(s * RPT, RPT)])
            plsc.subcore_barrier()
            for t in range(SW):
                pltpu.sync_copy(
                    src_hbm.at[(ci * NTILES + wid) * SW + t], src_v)
                pltpu.sync_copy(dst_hbm.at[wid * SW + t], dst_v)
                for b in range(2):
                    gather_sync(b, b)
                    scatter(b, b)

                def body(i, carry):
                    for b in range(2):
                        j = 2 * i + b
                        wait_scatter(j - 2, b)
                        gather_sync(j, b)
                        scatter(j, b)
                    return carry

                lax.fori_loop(1, SB // 2, body, 0)
                for b in range(2):
                    wait_scatter(SB - 2 + b, b)

            plsc.subcore_barrier()
            pltpu.sync_copy(
                acc.at[pl.ds(s * RPT, RPT)],
                out_hbm.at[pl.ds((2 * ci + c) * NP + s * RPT, RPT)])

    return prop


_prop1 = _make_prop(1)
_prop4 = _make_prop(N_EXP)


def _prep_body(x_ref, d0_ref, d1_ref, xs_ref, inv_ref):
    d = d0_ref[:, 0:1] + d1_ref[:, 0:1]
    inv = lax.rsqrt(jnp.maximum(d, 1.0))
    inv_b = jnp.broadcast_to(inv, (BLK, 128))
    inv_ref[...] = inv_b
    xs_ref[...] = x_ref[...] * inv_b


def _prep_call(x_pad, deg0, deg1):
    return pl.pallas_call(
        _prep_body,
        grid=(NP // BLK,),
        in_specs=[
            pl.BlockSpec((BLK, 128), lambda i: (i, 0)),
            pl.BlockSpec((BLK, DEG_W), lambda i: (i, 0)),
            pl.BlockSpec((BLK, DEG_W), lambda i: (i, 0)),
        ],
        out_specs=[
            pl.BlockSpec((BLK, 128), lambda i: (i, 0)),
            pl.BlockSpec((BLK, 128), lambda i: (i, 0)),
        ],
        out_shape=[
            jax.ShapeDtypeStruct((NP, 128), jnp.float32),
            jax.ShapeDtypeStruct((NP, 128), jnp.float32),
        ],
    )(x_pad, deg0, deg1)


def _experts_body(s0_ref, s1_ref, inv_ref, w1_ref, b1_ref, w2_ref, y4_ref):
    inv = inv_ref[...]
    px = (s0_ref[...] + s1_ref[...]) * inv
    for e in range(N_EXP):
        h = jnp.dot(px, w1_ref[e], preferred_element_type=jnp.float32)
        h = jnp.maximum(h + b1_ref[e:e + 1, :], 0.0)
        y = jnp.dot(h, w2_ref[e], preferred_element_type=jnp.float32)
        y4_ref[e] = y * inv


def _experts_call(s0, s1, inv_bc, W1, b1, W2):
    return pl.pallas_call(
        _experts_body,
        grid=(NP // BLK,),
        in_specs=[
            pl.BlockSpec((BLK, 128), lambda i: (i, 0)),
            pl.BlockSpec((BLK, 128), lambda i: (i, 0)),
            pl.BlockSpec((BLK, 128), lambda i: (i, 0)),
            pl.BlockSpec((N_EXP, D_IN, D_HID), lambda i: (0, 0, 0)),
            pl.BlockSpec((N_EXP, D_HID), lambda i: (0, 0)),
            pl.BlockSpec((N_EXP, D_HID, D_OUT), lambda i: (0, 0, 0)),
        ],
        out_specs=pl.BlockSpec((N_EXP, BLK, 128), lambda i: (0, i, 0)),
        out_shape=jax.ShapeDtypeStruct((N_EXP, NP, 128), jnp.float32),
    )(s0, s1, inv_bc, W1, b1, W2)


def _combine_body(x_ref, wg_ref, thr_ref, msk_ref, b2_ref, inv_ref, t_ref,
                  out_ref):
    logits = jnp.dot(x_ref[...], wg_ref[...],
                     preferred_element_type=jnp.float32)
    col = lax.broadcasted_iota(jnp.int32, (BLK, 128), 1)
    valid = col < N_EXP
    lm = jnp.where(valid, logits, -1e30)
    m = jnp.max(lm, axis=1, keepdims=True)
    ex = jnp.where(valid, jnp.exp(logits - m), 0.0)
    soft = ex / jnp.sum(ex, axis=1, keepdims=True)
    hard = 0.5 * (jnp.sign(logits - thr_ref[0:1, :]) + 1.0)
    g = soft * hard * msk_ref[0:1, :]
    g = g / (jnp.sum(g, axis=1, keepdims=True) + 1e-10)

    out = jnp.dot(g, b2_ref[...], preferred_element_type=jnp.float32)
    inv = inv_ref[...]
    for e in range(N_EXP):
        pe = (t_ref[2 * e] + t_ref[2 * e + 1]) * inv
        out = out + g[:, e:e + 1] * pe
    out_ref[...] = out


def _combine_call(x_pad, wg_pad, thr_bc, msk_bc, b2_pad, inv_bc, tstack):
    return pl.pallas_call(
        _combine_body,
        grid=(NP // BLK,),
        in_specs=[
            pl.BlockSpec((BLK, 128), lambda i: (i, 0)),
            pl.BlockSpec((128, 128), lambda i: (0, 0)),
            pl.BlockSpec((8, 128), lambda i: (0, 0)),
            pl.BlockSpec((8, 128), lambda i: (0, 0)),
            pl.BlockSpec((128, 128), lambda i: (0, 0)),
            pl.BlockSpec((BLK, 128), lambda i: (i, 0)),
            pl.BlockSpec((2 * N_EXP, BLK, 128), lambda i: (0, i, 0)),
        ],
        out_specs=pl.BlockSpec((BLK, 128), lambda i: (i, 0)),
        out_shape=jax.ShapeDtypeStruct((NP, 128), jnp.float32),
    )(x_pad, wg_pad, thr_bc, msk_bc, b2_pad, inv_bc, tstack)


def kernel(x, edge_index, w_gate, gate_threshold, W1, b1, W2, b2,
           experts_mask):
    f32 = jnp.float32
    x = x.astype(f32)
    src = edge_index[0].astype(jnp.int32)
    dst = edge_index[1].astype(jnp.int32)
    padn = EP - E
    fill = N + (jnp.arange(padn, dtype=jnp.int32) % (NP - N))
    src_flat = jnp.concatenate([src, fill])
    dst_flat = jnp.concatenate([dst, fill])
    dst_p = dst_flat.reshape(NTILES, NB, 128)
    src_p = src_flat.reshape(NTILES * SW, SB, 128)
    dst_q = dst_flat.reshape(NTILES * SW, SB, 128)
    shift = (jnp.arange(N_EXP, dtype=jnp.int32) * NP)[:, None]
    src4 = (src_flat[None, :] + shift).reshape(N_EXP * NTILES * SW, SB, 128)
    x_pad = jnp.pad(x, ((0, NP - N), (0, 0)))
    zeros128 = jnp.zeros((NP, 128), f32)
    zerosW = jnp.zeros((NP, DEG_W), f32)
    onesW = jnp.ones((128, DEG_W), f32)

    degp = _deg_kernel(dst_p, onesW, zerosW)
    xs, inv_bc = _prep_call(x_pad, degp[:NP], degp[NP:])
    sp = _prop1(xs, src_p, dst_q, zeros128)
    y4 = _experts_call(sp[:NP], sp[NP:], inv_bc,
                       W1.astype(f32), b1.astype(f32), W2.astype(f32))
    t4 = _prop4(y4.reshape(N_EXP * NP, 128), src4, dst_q, zeros128)
    tstack = t4.reshape(2 * N_EXP, NP, 128)

    wg_pad = jnp.zeros((128, 128), f32).at[:, :N_EXP].set(w_gate.astype(f32))
    thr_bc = jnp.zeros((8, 128), f32).at[:, :N_EXP].set(
        jnp.broadcast_to(gate_threshold.astype(f32), (8, N_EXP)))
    msk_bc = jnp.zeros((8, 128), f32).at[:, :N_EXP].set(
        jnp.broadcast_to(experts_mask.astype(f32), (8, N_EXP)))
    b2_pad = jnp.zeros((128, 128), f32).at[:N_EXP, :].set(b2.astype(f32))

    out = _combine_call(x_pad, wg_pad, thr_bc, msk_bc, b2_pad, inv_bc, tstack)
    return out[:N]


# speedup vs baseline: 3.0653x; 1.0030x over previous
import functools

import jax
import jax.numpy as jnp
from jax import lax
from jax.experimental import pallas as pl
from jax.experimental.pallas import tpu as pltpu
from jax.experimental.pallas import tpu_sc as plsc

N = 10000
NP = 10240
E = 320000
D_IN = 128
D_HID = 256
D_OUT = 128
N_EXP = 4

NTILES = 32
NB = 80
ET = NB * 128
SW = 2
SB = NB // SW
EP = NTILES * ET
DEG_W = 128
RPT = NP // 16

BLK = 256

_MESH = plsc.VectorSubcoreMesh(core_axis_name="c", subcore_axis_name="s")


@functools.partial(
    pl.kernel,
    mesh=_MESH,
    out_type=jax.ShapeDtypeStruct((2 * NP, DEG_W), jnp.float32),
    scratch_types=[
        pltpu.VMEM((NB, 128), jnp.int32),
        pltpu.VMEM((128, DEG_W), jnp.float32),
        pltpu.VMEM_SHARED((NP, DEG_W), jnp.float32),
        pltpu.SemaphoreType.DMA,
    ],
)
def _deg_kernel(dst_hbm, ones_hbm, zeros_hbm, out_hbm, dst_v, ones_v, acc,
                sem):
    c = lax.axis_index("c")
    s = lax.axis_index("s")
    wid = s * 2 + c
    pltpu.sync_copy(zeros_hbm.at[pl.ds(s * RPT, RPT)],
                    acc.at[pl.ds(s * RPT, RPT)])
    pltpu.sync_copy(ones_hbm, ones_v)
    pltpu.sync_copy(dst_hbm.at[wid], dst_v)
    plsc.subcore_barrier()

    def body(i, carry):
        for b in range(2):
            pltpu.async_copy(ones_v, acc.at[dst_v.at[2 * i + b]], sem,
                             add=True)
        for b in range(2):
            pltpu.make_async_copy(ones_v, acc.at[dst_v.at[2 * i + b]],
                                  sem).wait()
        return carry

    lax.fori_loop(0, NB // 2, body, 0)
    plsc.subcore_barrier()
    pltpu.sync_copy(acc.at[pl.ds(s * RPT, RPT)],
                    out_hbm.at[pl.ds(c * NP + s * RPT, RPT)])


def _make_prop(nchunks):
    @functools.partial(
        pl.kernel,
        mesh=_MESH,
        out_type=jax.ShapeDtypeStruct((nchunks * 2 * NP, 128), jnp.float32),
        scratch_types=[
            pltpu.VMEM((SB, 128), jnp.int32),
            pltpu.VMEM((SB, 128), jnp.int32),
            pltpu.VMEM((128, 128), jnp.float32),
            pltpu.VMEM((128, 128), jnp.float32),
            pltpu.VMEM_SHARED((NP, 128), jnp.float32),
            pltpu.SemaphoreType.DMA,
            pltpu.SemaphoreType.DMA,
            pltpu.SemaphoreType.DMA,
        ],
    )
    def prop(table_hbm, src_hbm, dst_hbm, zeros_hbm, out_hbm,
             src_v, dst_v, r0, r1, acc, sg, ss0, ss1):
        c = lax.axis_index("c")
        s = lax.axis_index("s")
        wid = s * 2 + c
        rows = [r0, r1]
        ss = [ss0, ss1]

        def gather(j, b):
            pltpu.async_copy(table_hbm.at[src_v.at[j]], rows[b], sg)

        def wait_gather(j, b):
            pltpu.make_async_copy(table_hbm.at[src_v.at[j]], rows[b],
                                  sg).wait()

        def scatter(j, b):
            pltpu.async_copy(rows[b], acc.at[dst_v.at[j]], ss[b], add=True)

        def wait_scatter(j, b):
            pltpu.make_async_copy(rows[b], acc.at[dst_v.at[j]], ss[b]).wait()

        for ci in range(nchunks):
            pltpu.sync_copy(zeros_hbm.at[pl.ds(s * RPT, RPT)],
                            acc.at[pl.ds(s * RPT, RPT)])
            plsc.subcore_barrier()
            for t in range(SW):
                pltpu.sync_copy(
                    src_hbm.at[(ci * NTILES + wid) * SW + t], src_v)
                pltpu.sync_copy(dst_hbm.at[wid * SW + t], dst_v)
                gather(0, 0)
                wait_gather(0, 0)
                gather(1, 1)
                scatter(0, 0)
                wait_gather(1, 1)
                wait_scatter(0, 0)
                gather(2, 0)
                scatter(1, 1)

                def body(i, carry):
                    for b in range(2):
                        j = 2 * i + b
                        wait_gather(j, b)
                        wait_scatter(j - 1, 1 - b)
                        gather(j + 1, 1 - b)
                        scatter(j, b)
                    return carry

                lax.fori_loop(1, SB // 2 - 1, body, 0)
                j = SB - 2
                wait_gather(j, 0)
                wait_scatter(j - 1, 1)
                gather(j + 1, 1)
                scatter(j, 0)
                wait_gather(j + 1, 1)
                wait_scatter(j, 0)
                scatter(j + 1, 1)
                wait_scatter(j + 1, 1)

            plsc.subcore_barrier()
            pltpu.sync_copy(
                acc.at[pl.ds(s * RPT, RPT)],
                out_hbm.at[pl.ds((2 * ci + c) * NP + s * RPT, RPT)])

    return prop


_prop1 = _make_prop(1)
_prop4 = _make_prop(N_EXP)


def _prep_body(x_ref, d0_ref, d1_ref, xs_ref, inv_ref):
    d = d0_ref[:, 0:1] + d1_ref[:, 0:1]
    inv = lax.rsqrt(jnp.maximum(d, 1.0))
    inv_b = jnp.broadcast_to(inv, (BLK, 128))
    inv_ref[...] = inv_b
    xs_ref[...] = x_ref[...] * inv_b


def _prep_call(x_pad, deg0, deg1):
    return pl.pallas_call(
        _prep_body,
        grid=(NP // BLK,),
        in_specs=[
            pl.BlockSpec((BLK, 128), lambda i: (i, 0)),
            pl.BlockSpec((BLK, DEG_W), lambda i: (i, 0)),
            pl.BlockSpec((BLK, DEG_W), lambda i: (i, 0)),
        ],
        out_specs=[
            pl.BlockSpec((BLK, 128), lambda i: (i, 0)),
            pl.BlockSpec((BLK, 128), lambda i: (i, 0)),
        ],
        out_shape=[
            jax.ShapeDtypeStruct((NP, 128), jnp.float32),
            jax.ShapeDtypeStruct((NP, 128), jnp.float32),
        ],
    )(x_pad, deg0, deg1)


def _experts_body(s0_ref, s1_ref, inv_ref, w1_ref, b1_ref, w2_ref, y4_ref):
    inv = inv_ref[...]
    px = (s0_ref[...] + s1_ref[...]) * inv
    for e in range(N_EXP):
        h = jnp.dot(px, w1_ref[e], preferred_element_type=jnp.float32)
        h = jnp.maximum(h + b1_ref[e:e + 1, :], 0.0)
        y = jnp.dot(h, w2_ref[e], preferred_element_type=jnp.float32)
        y4_ref[e] = y * inv


def _experts_call(s0, s1, inv_bc, W1, b1, W2):
    return pl.pallas_call(
        _experts_body,
        grid=(NP // BLK,),
        in_specs=[
            pl.BlockSpec((BLK, 128), lambda i: (i, 0)),
            pl.BlockSpec((BLK, 128), lambda i: (i, 0)),
            pl.BlockSpec((BLK, 128), lambda i: (i, 0)),
            pl.BlockSpec((N_EXP, D_IN, D_HID), lambda i: (0, 0, 0)),
            pl.BlockSpec((N_EXP, D_HID), lambda i: (0, 0)),
            pl.BlockSpec((N_EXP, D_HID, D_OUT), lambda i: (0, 0, 0)),
        ],
        out_specs=pl.BlockSpec((N_EXP, BLK, 128), lambda i: (0, i, 0)),
        out_shape=jax.ShapeDtypeStruct((N_EXP, NP, 128), jnp.float32),
    )(s0, s1, inv_bc, W1, b1, W2)


def _combine_body(x_ref, wg_ref, thr_ref, msk_ref, b2_ref, inv_ref, t_ref,
                  out_ref):
    logits = jnp.dot(x_ref[...], wg_ref[...],
                     preferred_element_type=jnp.float32)
    col = lax.broadcasted_iota(jnp.int32, (BLK, 128), 1)
    valid = col < N_EXP
    lm = jnp.where(valid, logits, -1e30)
    m = jnp.max(lm, axis=1, keepdims=True)
    ex = jnp.where(valid, jnp.exp(logits - m), 0.0)
    soft = ex / jnp.sum(ex, axis=1, keepdims=True)
    hard = 0.5 * (jnp.sign(logits - thr_ref[0:1, :]) + 1.0)
    g = soft * hard * msk_ref[0:1, :]
    g = g / (jnp.sum(g, axis=1, keepdims=True) + 1e-10)

    out = jnp.dot(g, b2_ref[...], preferred_element_type=jnp.float32)
    inv = inv_ref[...]
    for e in range(N_EXP):
        pe = (t_ref[2 * e] + t_ref[2 * e + 1]) * inv
        out = out + g[:, e:e + 1] * pe
    out_ref[...] = out


def _combine_call(x_pad, wg_pad, thr_bc, msk_bc, b2_pad, inv_bc, tstack):
    return pl.pallas_call(
        _combine_body,
        grid=(NP // BLK,),
        in_specs=[
            pl.BlockSpec((BLK, 128), lambda i: (i, 0)),
            pl.BlockSpec((128, 128), lambda i: (0, 0)),
            pl.BlockSpec((8, 128), lambda i: (0, 0)),
            pl.BlockSpec((8, 128), lambda i: (0, 0)),
            pl.BlockSpec((128, 128), lambda i: (0, 0)),
            pl.BlockSpec((BLK, 128), lambda i: (i, 0)),
            pl.BlockSpec((2 * N_EXP, BLK, 128), lambda i: (0, i, 0)),
        ],
        out_specs=pl.BlockSpec((BLK, 128), lambda i: (i, 0)),
        out_shape=jax.ShapeDtypeStruct((NP, 128), jnp.float32),
    )(x_pad, wg_pad, thr_bc, msk_bc, b2_pad, inv_bc, tstack)


def kernel(x, edge_index, w_gate, gate_threshold, W1, b1, W2, b2,
           experts_mask):
    f32 = jnp.float32
    x = x.astype(f32)
    src = edge_index[0].astype(jnp.int32)
    dst = edge_index[1].astype(jnp.int32)
    padn = EP - E
    fill = N + (jnp.arange(padn, dtype=jnp.int32) % (NP - N))
    src_flat = jnp.concatenate([src, fill])
    dst_flat = jnp.concatenate([dst, fill])
    dst_p = dst_flat.reshape(NTILES, NB, 128)
    src_p = src_flat.reshape(NTILES * SW, SB, 128)
    dst_q = dst_flat.reshape(NTILES * SW, SB, 128)
    shift = (jnp.arange(N_EXP, dtype=jnp.int32) * NP)[:, None]
    src4 = (src_flat[None, :] + shift).reshape(N_EXP * NTILES * SW, SB, 128)
    x_pad = jnp.pad(x, ((0, NP - N), (0, 0)))
    zeros128 = jnp.zeros((NP, 128), f32)
    zerosW = jnp.zeros((NP, DEG_W), f32)
    onesW = jnp.ones((128, DEG_W), f32)

    degp = _deg_kernel(dst_p, onesW, zerosW)
    xs, inv_bc = _prep_call(x_pad, degp[:NP], degp[NP:])
    sp = _prop1(xs, src_p, dst_q, zeros128)
    y4 = _experts_call(sp[:NP], sp[NP:], inv_bc,
                       W1.astype(f32), b1.astype(f32), W2.astype(f32))
    t4 = _prop4(y4.reshape(N_EXP * NP, 128), src4, dst_q, zeros128)
    tstack = t4.reshape(2 * N_EXP, NP, 128)

    wg_pad = jnp.zeros((128, 128), f32).at[:, :N_EXP].set(w_gate.astype(f32))
    thr_bc = jnp.zeros((8, 128), f32).at[:, :N_EXP].set(
        jnp.broadcast_to(gate_threshold.astype(f32), (8, N_EXP)))
    msk_bc = jnp.zeros((8, 128), f32).at[:, :N_EXP].set(
        jnp.broadcast_to(experts_mask.astype(f32), (8, N_EXP)))
    b2_pad = jnp.zeros((128, 128), f32).at[:N_EXP, :].set(b2.astype(f32))

    out = _combine_call(x_pad, wg_pad, thr_bc, msk_bc, b2_pad, inv_bc, tstack)
    return out[:N]
